# baseline (speedup 1.0000x reference)
import functools

import jax
import jax.numpy as jnp
from jax import lax
from jax.experimental import pallas as pl
from jax.experimental.pallas import tpu as pltpu
from jax.experimental.pallas import tpu_sc as plsc

P = 17040
NUM_WORKERS = 32
SG = 64
SG_PER_W = 9
CHUNK = SG * SG_PER_W
P_PAD = NUM_WORKERS * CHUNK
N = 1000
N_PAD = 1008
INF = 100000000.0
LANES = 16


def _tec_kernel(bx1_h, by1_h, bx2_h, by2_h, lab_h, xs_h, ys_h, ls_h, us_h,
                out_l, out_t, out_r, out_b, out_cls, out_cnt,
                bx1_v, by1_v, bx2_v, by2_v, lab_v, cidx_v,
                xs_v, ys_v, ls_v, us_v,
                l_v, t_v, r_v, b_v, cls_v, cnt_v):
    wid = lax.axis_index("s") * 2 + lax.axis_index("c")
    base = wid * CHUNK

    pltpu.sync_copy(bx1_h, bx1_v)
    pltpu.sync_copy(by1_h, by1_v)
    pltpu.sync_copy(bx2_h, bx2_v)
    pltpu.sync_copy(by2_h, by2_v)
    pltpu.sync_copy(lab_h, lab_v)
    pltpu.sync_copy(xs_h.at[pl.ds(base, CHUNK)], xs_v)
    pltpu.sync_copy(ys_h.at[pl.ds(base, CHUNK)], ys_v)
    pltpu.sync_copy(ls_h.at[pl.ds(base, CHUNK)], ls_v)
    pltpu.sync_copy(us_h.at[pl.ds(base, CHUNK)], us_v)

    def _lanered(v, op):
        x = [v[k] for k in range(LANES)]
        while len(x) > 1:
            x = [op(x[i], x[i + 1]) for i in range(0, len(x) - 1, 2)] \
                + ([x[-1]] if len(x) % 2 else [])
        return x[0]

    def minmax4(ref, sbase):
        a = ref[pl.ds(sbase, LANES)]
        b = ref[pl.ds(sbase + 16, LANES)]
        c = ref[pl.ds(sbase + 32, LANES)]
        d = ref[pl.ds(sbase + 48, LANES)]
        lo = jnp.minimum(jnp.minimum(a, b), jnp.minimum(c, d))
        hi = jnp.maximum(jnp.maximum(a, b), jnp.maximum(c, d))
        return _lanered(lo, jnp.minimum), _lanered(hi, jnp.maximum)

    def do_sg(s, _):
        sbase = s * SG
        xmn, xmx = minmax4(xs_v, sbase)
        ymn, ymx = minmax4(ys_v, sbase)
        lmn, _ = minmax4(ls_v, sbase)
        _, umx = minmax4(us_v, sbase)
        tx1 = xmx + 1.0
        tx2 = xmn - 1.0
        ty1 = ymx + 1.0
        ty2 = ymn - 1.0
        tsz = 2.0 * umx + 1.0
        tl = lmn - 1.0
        wx1 = xmn - umx - 1.0
        wx2 = xmx + umx + 1.0
        wy1 = ymn - umx - 1.0
        wy2 = ymx + umx + 1.0

        def do_filt(bg, pos):
            boff = bg * LANES
            x1g = bx1_v[pl.ds(boff, LANES)]
            y1g = by1_v[pl.ds(boff, LANES)]
            x2g = bx2_v[pl.ds(boff, LANES)]
            y2g = by2_v[pl.ds(boff, LANES)]
            bw = x2g - x1g
            bh = y2g - y1g
            keep = ((x1g <= tx1) & (x2g >= tx2) &
                    (y1g <= ty1) & (y2g >= ty2) &
                    (x1g >= wx1) & (x2g <= wx2) &
                    (y1g >= wy1) & (y2g <= wy2) &
                    (bw <= tsz) & (bh <= tsz) &
                    (jnp.maximum(bw, bh) >= tl))
            idxv = lax.broadcasted_iota(jnp.int32, (LANES,), 0) + boff
            plsc.store_compressed(cidx_v.at[pl.ds(pos, LANES)], idxv,
                                  mask=keep)
            return pos + plsc.all_reduce_population_count(keep)[0]

        pos = lax.fori_loop(0, N_PAD // LANES, do_filt, 0)
        cidx_v[pl.ds(pos, LANES)] = jnp.full((LANES,), N, jnp.int32)
        nbg = (pos + 15) >> 4

        for gp in range(SG // (2 * LANES)):
            off = sbase + gp * 2 * LANES
            pxa = xs_v[pl.ds(off, LANES)]
            pya = ys_v[pl.ds(off, LANES)]
            prla = ls_v[pl.ds(off, LANES)]
            prua = us_v[pl.ds(off, LANES)]
            pxb = xs_v[pl.ds(off + LANES, LANES)]
            pyb = ys_v[pl.ds(off + LANES, LANES)]
            prlb = ls_v[pl.ds(off + LANES, LANES)]
            prub = us_v[pl.ds(off + LANES, LANES)]

            def do_bg(bg, carry, pxa=pxa, pya=pya, prla=prla, prua=prua,
                      pxb=pxb, pyb=pyb, prlb=prlb, prub=prub):
                bidx = cidx_v[pl.ds(bg * LANES, LANES)]
                x1g = plsc.load_gather(bx1_v, [bidx])
                y1g = plsc.load_gather(by1_v, [bidx])
                x2g = plsc.load_gather(bx2_v, [bidx])
                y2g = plsc.load_gather(by2_v, [bidx])
                for k in range(LANES):
                    baa, bia, bab, bib = carry
                    x1 = x1g[k]
                    y1 = y1g[k]
                    x2 = x2g[k]
                    y2 = y2g[k]
                    bi = bidx[k]
                    la = pxa - x1
                    ta = pya - y1
                    ra = x2 - pxa
                    bba = y2 - pya
                    areaa = (la + ra) * (ta + bba)
                    dmna = jnp.minimum(jnp.minimum(la, ta),
                                       jnp.minimum(ra, bba))
                    dmxa = jnp.maximum(jnp.maximum(la, ta),
                                       jnp.maximum(ra, bba))
                    upda = ((dmna > 0.0) & (prla <= dmxa) & (dmxa <= prua)
                            & (areaa < baa))
                    lb = pxb - x1
                    tb = pyb - y1
                    rb = x2 - pxb
                    bbb = y2 - pyb
                    areab = (lb + rb) * (tb + bbb)
                    dmnb = jnp.minimum(jnp.minimum(lb, tb),
                                       jnp.minimum(rb, bbb))
                    dmxb = jnp.maximum(jnp.maximum(lb, tb),
                                       jnp.maximum(rb, bbb))
                    updb = ((dmnb > 0.0) & (prlb <= dmxb) & (dmxb <= prub)
                            & (areab < bab))
                    carry = (jnp.where(upda, areaa, baa),
                             jnp.where(upda, bi, bia),
                             jnp.where(updb, areab, bab),
                             jnp.where(updb, bi, bib))
                return carry

            init = (jnp.full((LANES,), INF, jnp.float32),
                    jnp.zeros((LANES,), jnp.int32),
                    jnp.full((LANES,), INF, jnp.float32),
                    jnp.zeros((LANES,), jnp.int32))
            baa, bia, bab, bib = lax.fori_loop(0, nbg, do_bg, init)

            for (goff, px, py, ba, bi) in ((off, pxa, pya, baa, bia),
                                           (off + LANES, pxb, pyb, bab, bib)):
                gx1 = plsc.load_gather(bx1_v, [bi])
                gy1 = plsc.load_gather(by1_v, [bi])
                gx2 = plsc.load_gather(bx2_v, [bi])
                gy2 = plsc.load_gather(by2_v, [bi])
                glab = plsc.load_gather(lab_v, [bi])
                bl = px - gx1
                bt = py - gy1
                br = gx2 - px
                bb = gy2 - py
                cls = jnp.where(ba == INF, 0, glab)
                r0 = jnp.minimum(bl, bt) / jnp.maximum(bl, bt)
                r1 = jnp.minimum(br, bb) / jnp.maximum(br, bb)
                prod = r0 * r1
                seed = ((lax.bitcast_convert_type(prod, jnp.int32) >> 1)
                        + 0x1FBD1DF5)
                y = lax.bitcast_convert_type(seed, jnp.float32)
                for _ in range(4):
                    y = 0.5 * (y + prod / y)
                cnt = jnp.where(prod < 0.0, jnp.float32(jnp.nan), y)

                l_v[pl.ds(goff, LANES)] = bl
                t_v[pl.ds(goff, LANES)] = bt
                r_v[pl.ds(goff, LANES)] = br
                b_v[pl.ds(goff, LANES)] = bb
                cls_v[pl.ds(goff, LANES)] = cls
                cnt_v[pl.ds(goff, LANES)] = cnt
        return 0

    lax.fori_loop(0, SG_PER_W, do_sg, 0)

    pltpu.sync_copy(l_v, out_l.at[pl.ds(base, CHUNK)])
    pltpu.sync_copy(t_v, out_t.at[pl.ds(base, CHUNK)])
    pltpu.sync_copy(r_v, out_r.at[pl.ds(base, CHUNK)])
    pltpu.sync_copy(b_v, out_b.at[pl.ds(base, CHUNK)])
    pltpu.sync_copy(cls_v, out_cls.at[pl.ds(base, CHUNK)])
    pltpu.sync_copy(cnt_v, out_cnt.at[pl.ds(base, CHUNK)])


@functools.partial(
    pl.kernel,
    out_type=(
        jax.ShapeDtypeStruct((P_PAD,), jnp.float32),
        jax.ShapeDtypeStruct((P_PAD,), jnp.float32),
        jax.ShapeDtypeStruct((P_PAD,), jnp.float32),
        jax.ShapeDtypeStruct((P_PAD,), jnp.float32),
        jax.ShapeDtypeStruct((P_PAD,), jnp.int32),
        jax.ShapeDtypeStruct((P_PAD,), jnp.float32),
    ),
    mesh=plsc.VectorSubcoreMesh(core_axis_name="c", subcore_axis_name="s"),
    compiler_params=pltpu.CompilerParams(needs_layout_passes=False),
    scratch_types=[
        pltpu.VMEM((N_PAD,), jnp.float32),
        pltpu.VMEM((N_PAD,), jnp.float32),
        pltpu.VMEM((N_PAD,), jnp.float32),
        pltpu.VMEM((N_PAD,), jnp.float32),
        pltpu.VMEM((N_PAD,), jnp.int32),
        pltpu.VMEM((N_PAD + LANES,), jnp.int32),
        pltpu.VMEM((CHUNK,), jnp.float32),
        pltpu.VMEM((CHUNK,), jnp.float32),
        pltpu.VMEM((CHUNK,), jnp.float32),
        pltpu.VMEM((CHUNK,), jnp.float32),
        pltpu.VMEM((CHUNK,), jnp.float32),
        pltpu.VMEM((CHUNK,), jnp.float32),
        pltpu.VMEM((CHUNK,), jnp.float32),
        pltpu.VMEM((CHUNK,), jnp.float32),
        pltpu.VMEM((CHUNK,), jnp.int32),
        pltpu.VMEM((CHUNK,), jnp.float32),
    ],
)
def _sc_assign(*refs):
    _tec_kernel(*refs)


def _interleave(a):
    return a.reshape(SG_PER_W, NUM_WORKERS, SG).transpose(1, 0, 2).reshape(-1)


def _deinterleave(a):
    return a.reshape(NUM_WORKERS, SG_PER_W, SG).transpose(1, 0, 2).reshape(-1)


def kernel(bboxes, labels, all_points, all_regress_ranges):
    bx1 = jnp.pad(bboxes[:, 0], (0, N_PAD - N))
    by1 = jnp.pad(bboxes[:, 1], (0, N_PAD - N))
    bx2 = jnp.pad(bboxes[:, 2], (0, N_PAD - N))
    by2 = jnp.pad(bboxes[:, 3], (0, N_PAD - N))
    lab = jnp.pad(labels, (0, N_PAD - N))
    xs = _interleave(jnp.pad(all_points[:, 0], (0, P_PAD - P)))
    ys = _interleave(jnp.pad(all_points[:, 1], (0, P_PAD - P)))
    ls = _interleave(jnp.pad(all_regress_ranges[:, 0], (0, P_PAD - P)))
    us = _interleave(jnp.pad(all_regress_ranges[:, 1], (0, P_PAD - P)))

    l, t, r, b, cls, cnt = _sc_assign(bx1, by1, bx2, by2, lab,
                                      xs, ys, ls, us)
    l = _deinterleave(l)[:P]
    t = _deinterleave(t)[:P]
    r = _deinterleave(r)[:P]
    b = _deinterleave(b)[:P]
    reg_targets = jnp.stack([l, t, r, b], axis=1)
    return reg_targets, _deinterleave(cls)[:P], _deinterleave(cnt)[:P, None]


# speedup vs baseline: 1.8273x; 1.8273x over previous
import functools

import jax
import jax.numpy as jnp
from jax import lax
from jax.experimental import pallas as pl
from jax.experimental.pallas import tpu as pltpu
from jax.experimental.pallas import tpu_sc as plsc

P = 17040
NUM_WORKERS = 32
SG = 64
SG_PER_W = 9
CHUNK = SG * SG_PER_W
P_PAD = NUM_WORKERS * CHUNK
N = 1000
N_PAD = 1008
INF = 100000000.0
LANES = 16


def _tec_kernel(bx1_h, by1_h, bx2_h, by2_h, lab_h, xs_h, ys_h, ls_h, us_h,
                out_l, out_t, out_r, out_b, out_cls, out_cnt,
                bx1_v, by1_v, bx2_v, by2_v, lab_v, cidx_v,
                xs_v, ys_v, ls_v, us_v,
                l_v, t_v, r_v, b_v, cls_v, cnt_v):
    wid = lax.axis_index("s") * 2 + lax.axis_index("c")
    base = wid * CHUNK

    pltpu.sync_copy(bx1_h, bx1_v)
    pltpu.sync_copy(by1_h, by1_v)
    pltpu.sync_copy(bx2_h, bx2_v)
    pltpu.sync_copy(by2_h, by2_v)
    pltpu.sync_copy(lab_h, lab_v)
    pltpu.sync_copy(xs_h.at[pl.ds(base, CHUNK)], xs_v)
    pltpu.sync_copy(ys_h.at[pl.ds(base, CHUNK)], ys_v)
    pltpu.sync_copy(ls_h.at[pl.ds(base, CHUNK)], ls_v)
    pltpu.sync_copy(us_h.at[pl.ds(base, CHUNK)], us_v)

    def _lanered(v, op):
        x = [v[k] for k in range(LANES)]
        while len(x) > 1:
            x = [op(x[i], x[i + 1]) for i in range(0, len(x) - 1, 2)] \
                + ([x[-1]] if len(x) % 2 else [])
        return x[0]

    def minmax4(ref, sbase):
        a = ref[pl.ds(sbase, LANES)]
        b = ref[pl.ds(sbase + 16, LANES)]
        c = ref[pl.ds(sbase + 32, LANES)]
        d = ref[pl.ds(sbase + 48, LANES)]
        lo = jnp.minimum(jnp.minimum(a, b), jnp.minimum(c, d))
        hi = jnp.maximum(jnp.maximum(a, b), jnp.maximum(c, d))
        return _lanered(lo, jnp.minimum), _lanered(hi, jnp.maximum)

    def do_sg(s, _):
        sbase = s * SG
        xmn, xmx = minmax4(xs_v, sbase)
        ymn, ymx = minmax4(ys_v, sbase)
        lmn, _ = minmax4(ls_v, sbase)
        _, umx = minmax4(us_v, sbase)
        tx1 = xmx + 1.0
        tx2 = xmn - 1.0
        ty1 = ymx + 1.0
        ty2 = ymn - 1.0
        tsz = 2.0 * umx + 1.0
        tl = lmn - 1.0
        wx1 = xmn - umx - 1.0
        wx2 = xmx + umx + 1.0
        wy1 = ymn - umx - 1.0
        wy2 = ymx + umx + 1.0

        def do_filt(bg, pos):
            boff = bg * LANES
            x1g = bx1_v[pl.ds(boff, LANES)]
            y1g = by1_v[pl.ds(boff, LANES)]
            x2g = bx2_v[pl.ds(boff, LANES)]
            y2g = by2_v[pl.ds(boff, LANES)]
            bw = x2g - x1g
            bh = y2g - y1g
            keep = ((x1g <= tx1) & (x2g >= tx2) &
                    (y1g <= ty1) & (y2g >= ty2) &
                    (x1g >= wx1) & (x2g <= wx2) &
                    (y1g >= wy1) & (y2g <= wy2) &
                    (bw <= tsz) & (bh <= tsz) &
                    (jnp.maximum(bw, bh) >= tl))
            idxv = lax.broadcasted_iota(jnp.int32, (LANES,), 0) + boff
            plsc.store_compressed(cidx_v.at[pl.ds(pos, LANES)], idxv,
                                  mask=keep)
            return pos + plsc.all_reduce_population_count(keep)[0]

        pos = lax.fori_loop(0, N_PAD // LANES, do_filt, 0)
        cidx_v[pl.ds(pos, LANES)] = jnp.full((LANES,), N, jnp.int32)
        nbg = (pos + 15) >> 4

        for gp in range(SG // LANES):
            off = sbase + gp * LANES
            pxa = xs_v[pl.ds(off, LANES)]
            pya = ys_v[pl.ds(off, LANES)]
            prla = ls_v[pl.ds(off, LANES)]
            prua = us_v[pl.ds(off, LANES)]

            def do_bg(bg, carry, pxa=pxa, pya=pya, prla=prla, prua=prua):
                bidx = cidx_v[pl.ds(bg * LANES, LANES)]
                x1g = plsc.load_gather(bx1_v, [bidx])
                y1g = plsc.load_gather(by1_v, [bidx])
                x2g = plsc.load_gather(bx2_v, [bidx])
                y2g = plsc.load_gather(by2_v, [bidx])
                for k in range(LANES):
                    baa, bia = carry
                    x1 = x1g[k]
                    y1 = y1g[k]
                    x2 = x2g[k]
                    y2 = y2g[k]
                    bi = bidx[k]
                    la = pxa - x1
                    ta = pya - y1
                    ra = x2 - pxa
                    bba = y2 - pya
                    areaa = (la + ra) * (ta + bba)
                    dmna = jnp.minimum(jnp.minimum(la, ta),
                                       jnp.minimum(ra, bba))
                    dmxa = jnp.maximum(jnp.maximum(la, ta),
                                       jnp.maximum(ra, bba))
                    upda = ((dmna > 0.0) & (prla <= dmxa) & (dmxa <= prua)
                            & (areaa < baa))
                    carry = (jnp.where(upda, areaa, baa),
                             jnp.where(upda, bi, bia))
                return carry

            init = (jnp.full((LANES,), INF, jnp.float32),
                    jnp.zeros((LANES,), jnp.int32))
            baa, bia = lax.fori_loop(0, nbg, do_bg, init)

            for (goff, px, py, ba, bi) in ((off, pxa, pya, baa, bia),):
                gx1 = plsc.load_gather(bx1_v, [bi])
                gy1 = plsc.load_gather(by1_v, [bi])
                gx2 = plsc.load_gather(bx2_v, [bi])
                gy2 = plsc.load_gather(by2_v, [bi])
                glab = plsc.load_gather(lab_v, [bi])
                bl = px - gx1
                bt = py - gy1
                br = gx2 - px
                bb = gy2 - py
                cls = jnp.where(ba == INF, 0, glab)
                r0 = jnp.minimum(bl, bt) / jnp.maximum(bl, bt)
                r1 = jnp.minimum(br, bb) / jnp.maximum(br, bb)
                prod = r0 * r1
                seed = ((lax.bitcast_convert_type(prod, jnp.int32) >> 1)
                        + 0x1FBD1DF5)
                y = lax.bitcast_convert_type(seed, jnp.float32)
                for _ in range(4):
                    y = 0.5 * (y + prod / y)
                cnt = jnp.where(prod < 0.0, jnp.float32(jnp.nan), y)

                l_v[pl.ds(goff, LANES)] = bl
                t_v[pl.ds(goff, LANES)] = bt
                r_v[pl.ds(goff, LANES)] = br
                b_v[pl.ds(goff, LANES)] = bb
                cls_v[pl.ds(goff, LANES)] = cls
                cnt_v[pl.ds(goff, LANES)] = cnt
        return 0

    lax.fori_loop(0, SG_PER_W, do_sg, 0)

    pltpu.sync_copy(l_v, out_l.at[pl.ds(base, CHUNK)])
    pltpu.sync_copy(t_v, out_t.at[pl.ds(base, CHUNK)])
    pltpu.sync_copy(r_v, out_r.at[pl.ds(base, CHUNK)])
    pltpu.sync_copy(b_v, out_b.at[pl.ds(base, CHUNK)])
    pltpu.sync_copy(cls_v, out_cls.at[pl.ds(base, CHUNK)])
    pltpu.sync_copy(cnt_v, out_cnt.at[pl.ds(base, CHUNK)])


@functools.partial(
    pl.kernel,
    out_type=(
        jax.ShapeDtypeStruct((P_PAD,), jnp.float32),
        jax.ShapeDtypeStruct((P_PAD,), jnp.float32),
        jax.ShapeDtypeStruct((P_PAD,), jnp.float32),
        jax.ShapeDtypeStruct((P_PAD,), jnp.float32),
        jax.ShapeDtypeStruct((P_PAD,), jnp.int32),
        jax.ShapeDtypeStruct((P_PAD,), jnp.float32),
    ),
    mesh=plsc.VectorSubcoreMesh(core_axis_name="c", subcore_axis_name="s"),
    compiler_params=pltpu.CompilerParams(needs_layout_passes=False),
    scratch_types=[
        pltpu.VMEM((N_PAD,), jnp.float32),
        pltpu.VMEM((N_PAD,), jnp.float32),
        pltpu.VMEM((N_PAD,), jnp.float32),
        pltpu.VMEM((N_PAD,), jnp.float32),
        pltpu.VMEM((N_PAD,), jnp.int32),
        pltpu.VMEM((N_PAD + LANES,), jnp.int32),
        pltpu.VMEM((CHUNK,), jnp.float32),
        pltpu.VMEM((CHUNK,), jnp.float32),
        pltpu.VMEM((CHUNK,), jnp.float32),
        pltpu.VMEM((CHUNK,), jnp.float32),
        pltpu.VMEM((CHUNK,), jnp.float32),
        pltpu.VMEM((CHUNK,), jnp.float32),
        pltpu.VMEM((CHUNK,), jnp.float32),
        pltpu.VMEM((CHUNK,), jnp.float32),
        pltpu.VMEM((CHUNK,), jnp.int32),
        pltpu.VMEM((CHUNK,), jnp.float32),
    ],
)
def _sc_assign(*refs):
    _tec_kernel(*refs)


def _interleave(a):
    return a.reshape(SG_PER_W, NUM_WORKERS, SG).transpose(1, 0, 2).reshape(-1)


def _deinterleave(a):
    return a.reshape(NUM_WORKERS, SG_PER_W, SG).transpose(1, 0, 2).reshape(-1)


def kernel(bboxes, labels, all_points, all_regress_ranges):
    bx1 = jnp.pad(bboxes[:, 0], (0, N_PAD - N))
    by1 = jnp.pad(bboxes[:, 1], (0, N_PAD - N))
    bx2 = jnp.pad(bboxes[:, 2], (0, N_PAD - N))
    by2 = jnp.pad(bboxes[:, 3], (0, N_PAD - N))
    lab = jnp.pad(labels, (0, N_PAD - N))
    xs = _interleave(jnp.pad(all_points[:, 0], (0, P_PAD - P)))
    ys = _interleave(jnp.pad(all_points[:, 1], (0, P_PAD - P)))
    ls = _interleave(jnp.pad(all_regress_ranges[:, 0], (0, P_PAD - P)))
    us = _interleave(jnp.pad(all_regress_ranges[:, 1], (0, P_PAD - P)))

    l, t, r, b, cls, cnt = _sc_assign(bx1, by1, bx2, by2, lab,
                                      xs, ys, ls, us)
    l = _deinterleave(l)[:P]
    t = _deinterleave(t)[:P]
    r = _deinterleave(r)[:P]
    b = _deinterleave(b)[:P]
    reg_targets = jnp.stack([l, t, r, b], axis=1)
    return reg_targets, _deinterleave(cls)[:P], _deinterleave(cnt)[:P, None]
